# baseline (speedup 1.0000x reference)
import jax
import jax.numpy as jnp
from jax.experimental import pallas as pl
from jax.experimental.pallas import tpu as pltpu

B = 64
NB = 128
HID = 64
K = 8192
S = 64 * 64
BB = 8
NSTEPS = B // BB
BETA = 0.25


def _fused_kernel(x_ref, w1_ref, b1_ref, m2_ref, b2_ref, m3_ref, b3_ref,
                  m4_ref, b4_ref, c_ref, ct_ref, wdt_ref, bd_ref,
                  recon_ref, q_ref, idx_ref, loss_ref, rl_ref, vl_ref,
                  xm_ref):
    i = pl.program_id(0)
    part = jnp.sum(x_ref[...], axis=-1) * (1.0 / S)
    xm_ref[pl.ds(i * BB, BB), :] = part

    @pl.when(i == NSTEPS - 1)
    def _tail():
        xm = xm_ref[...]

        row = jax.lax.broadcasted_iota(jnp.int32, (B * NB, 1), 0)
        l_id = jax.lax.rem(row, NB)
        at_first = l_id == 0
        at_last = l_id == NB - 1

        selr = jax.lax.broadcasted_iota(jnp.int32, (B * NB, B), 0)
        selb = jax.lax.broadcasted_iota(jnp.int32, (B * NB, B), 1)
        sel = (selr // NB == selb).astype(jnp.float32)
        rows_xm = jnp.dot(sel, xm,
                          preferred_element_type=jnp.float32)
        lane_nb = jax.lax.broadcasted_iota(jnp.int32, (B * NB, NB), 1)
        h0 = jnp.sum(jnp.where(lane_nb == l_id, rows_xm, 0.0),
                     axis=1, keepdims=True)

        def shift_prev(h):
            z = jnp.zeros((1, h.shape[1]), jnp.float32)
            s = jnp.concatenate([z, h[:-1, :]], axis=0)
            return jnp.where(at_first, 0.0, s)

        def shift_next(h):
            z = jnp.zeros((1, h.shape[1]), jnp.float32)
            s = jnp.concatenate([h[1:, :], z], axis=0)
            return jnp.where(at_last, 0.0, s)

        w1 = w1_ref[...]
        h = (shift_prev(h0) * w1[0:1, :]
             + h0 * w1[1:2, :]
             + shift_next(h0) * w1[2:3, :]
             + b1_ref[...])
        h = jnp.maximum(h, 0.0)

        for m_ref, b_ref in ((m2_ref, b2_ref), (m3_ref, b3_ref),
                             (m4_ref, b4_ref)):
            h3 = jnp.concatenate(
                [shift_prev(h), h, shift_next(h)], axis=1)
            h = jnp.dot(h3, m_ref[...],
                        preferred_element_type=jnp.float32) + b_ref[...]
            h = jnp.maximum(h, 0.0)

        segc = jax.lax.broadcasted_iota(jnp.int32, (B, B * NB), 1)
        segr = jax.lax.broadcasted_iota(jnp.int32, (B, B * NB), 0)
        selt = (segc // NB == segr).astype(jnp.float32)
        z = jnp.dot(selt, h,
                    preferred_element_type=jnp.float32) * (1.0 / NB)

        ct = ct_ref[...]
        zz = jnp.sum(z * z, axis=1, keepdims=True)
        zc = jnp.dot(z, ct, preferred_element_type=jnp.float32)
        c2 = jnp.sum(ct * ct, axis=0, keepdims=True)
        d = zz - 2.0 * zc + c2
        dmin = jnp.min(d, axis=1, keepdims=True)
        lane = jax.lax.broadcasted_iota(jnp.int32, (B, K), 1)
        idx = jnp.min(jnp.where(d == dmin, lane, K), axis=1,
                      keepdims=True)
        onehot = (lane == idx).astype(jnp.float32)
        q = jnp.dot(onehot, c_ref[...],
                    preferred_element_type=jnp.float32)

        recon = jnp.dot(q, wdt_ref[...],
                        preferred_element_type=jnp.float32) + bd_ref[...]
        se = (recon - xm) ** 2
        rl = jnp.sum(jnp.sum(se, axis=1, keepdims=True), axis=0,
                     keepdims=True) * (1.0 / (B * NB))
        qe = (q - z) ** 2
        vl = jnp.sum(jnp.sum(qe, axis=1, keepdims=True), axis=0,
                     keepdims=True) * ((1.0 + BETA) / (B * HID))

        recon_ref[...] = recon
        q_ref[...] = q
        idx_ref[...] = idx
        rl_ref[...] = rl
        vl_ref[...] = vl
        loss_ref[...] = rl + vl


def kernel(x, W1, b1, W2, b2, W3, b3, W4, b4, codebook, Wd, bd):
    x3 = x.reshape(B, NB, S)
    w1m = jnp.transpose(W1, (2, 1, 0)).reshape(3, HID)
    m2 = jnp.transpose(W2, (2, 1, 0)).reshape(3 * HID, HID)
    m3 = jnp.transpose(W3, (2, 1, 0)).reshape(3 * HID, HID)
    m4 = jnp.transpose(W4, (2, 1, 0)).reshape(3 * HID, HID)
    ct = codebook.T
    wdt = Wd.T
    b1r, b2r, b3r, b4r = (v.reshape(1, HID) for v in (b1, b2, b3, b4))
    bdr = bd.reshape(1, NB)

    full = lambda shape: pl.BlockSpec(shape, lambda i: (0,) * len(shape))
    out_shapes = (
        jax.ShapeDtypeStruct((B, NB), jnp.float32),
        jax.ShapeDtypeStruct((B, HID), jnp.float32),
        jax.ShapeDtypeStruct((B, 1), jnp.int32),
        jax.ShapeDtypeStruct((1, 1), jnp.float32),
        jax.ShapeDtypeStruct((1, 1), jnp.float32),
        jax.ShapeDtypeStruct((1, 1), jnp.float32),
    )
    recon, q, idx, loss, rl, vl = pl.pallas_call(
        _fused_kernel,
        grid=(NSTEPS,),
        in_specs=[
            pl.BlockSpec((BB, NB, S), lambda i: (i, 0, 0)),
            full((3, HID)), full((1, HID)),
            full((3 * HID, HID)), full((1, HID)),
            full((3 * HID, HID)), full((1, HID)),
            full((3 * HID, HID)), full((1, HID)),
            full((K, HID)), full((HID, K)),
            full((HID, NB)), full((1, NB)),
        ],
        out_specs=(
            full((B, NB)), full((B, HID)), full((B, 1)),
            full((1, 1)), full((1, 1)), full((1, 1)),
        ),
        out_shape=out_shapes,
        scratch_shapes=[pltpu.VMEM((B, NB), jnp.float32)],
    )(x3, w1m, b1r, m2, b2r, m3, b3r, m4, b4r, codebook, ct, wdt, bdr)

    return (recon, q[:, None, :], idx, loss[0, 0], rl[0, 0], vl[0, 0])
